# Initial kernel scaffold; baseline (speedup 1.0000x reference)
import jax
import jax.numpy as jnp
from jax.experimental import pallas as pl


def kernel(cls_out_0, cls_out_1, cls_out_2, cls_out_3, cls_out_4, box_out_0, box_out_1, box_out_2, box_out_3, box_out_4, anchor_boxes):
    raise NotImplementedError("write your pallas kernel here")


# speedup vs baseline: 2.6969x; 2.6969x over previous
import jax
import jax.numpy as jnp
from jax.experimental import pallas as pl

_NUM_CLASSES = 90
_N = 5000
_NP = 5120
_NMAX = 100
_IOU = 0.5


def _det_kernel(rel_ref, anc_ref, cls_ref, box_ref, keep_ref):
    rel = rel_ref[0]
    anc = anc_ref[0]
    clsf = cls_ref[0]
    ty = rel[0:1]
    tx = rel[1:2]
    th = rel[2:3]
    tw = rel[3:4]
    a0 = anc[0:1]
    a1 = anc[1:2]
    a2 = anc[2:3]
    a3 = anc[3:4]
    yca = (a0 + a2) * 0.5
    xca = (a1 + a3) * 0.5
    ha = a2 - a0
    wa = a3 - a1
    w = jnp.exp(tw) * wa
    h = jnp.exp(th) * ha
    yc = ty * ha + yca
    xc = tx * wa + xca
    xmin = xc - w * 0.5
    ymin = yc - h * 0.5
    xmax = xc + w * 0.5
    ymax = yc + h * 0.5

    pos = jax.lax.broadcasted_iota(jnp.int32, (1, _NP), 1)
    validm = pos < _N
    neg = jnp.float32(-1e30)
    mx = jnp.maximum(
        jnp.maximum(jnp.max(jnp.where(validm, xmin, neg)),
                    jnp.max(jnp.where(validm, ymin, neg))),
        jnp.maximum(jnp.max(jnp.where(validm, xmax, neg)),
                    jnp.max(jnp.where(validm, ymax, neg))))
    off = clsf * (mx + 1.0)
    bx0 = xmin + off
    by0 = ymin + off
    bx1 = xmax + off
    by1 = ymax + off
    area = (bx1 - bx0) * (by1 - by0)
    keep0 = validm.astype(jnp.float32)
    stacked = jnp.concatenate([bx0, by0, bx1, by1, area], axis=0)

    def body(i, keep):
        m = pos == i
        vals = jnp.max(jnp.where(m, stacked, neg), axis=1, keepdims=True)
        xi0 = vals[0:1]
        yi0 = vals[1:2]
        xi1 = vals[2:3]
        yi1 = vals[3:4]
        ai = vals[4:5]
        ki = jnp.max(jnp.where(m, keep, 0.0))
        iw = jnp.maximum(jnp.minimum(bx1, xi1) - jnp.maximum(bx0, xi0), 0.0)
        ih = jnp.maximum(jnp.minimum(by1, yi1) - jnp.maximum(by0, yi0), 0.0)
        inter = iw * ih
        iou = inter / jnp.maximum(area + ai - inter, 1e-9)
        sup = (iou > _IOU) & (pos > i) & (ki > 0.0)
        return jnp.where(sup, 0.0, keep)

    keep = jax.lax.fori_loop(0, _N, body, keep0)
    box_ref[0] = jnp.concatenate([xmin, ymin, xmax, ymax], axis=0)
    keep_ref[0] = keep


@jax.jit
def kernel(cls_out_0, cls_out_1, cls_out_2, cls_out_3, cls_out_4,
           box_out_0, box_out_1, box_out_2, box_out_3, box_out_4,
           anchor_boxes):
    cls_outputs = [cls_out_0, cls_out_1, cls_out_2, cls_out_3, cls_out_4]
    box_outputs = [box_out_0, box_out_1, box_out_2, box_out_3, box_out_4]
    B = cls_out_0.shape[0]
    cls_all = jnp.concatenate(
        [jnp.transpose(c, (0, 2, 3, 1)).reshape(B, -1, _NUM_CLASSES)
         for c in cls_outputs], axis=1)
    box_all = jnp.concatenate(
        [jnp.transpose(b, (0, 2, 3, 1)).reshape(B, -1, 4)
         for b in box_outputs], axis=1)
    topv, topk_idx = jax.lax.top_k(cls_all.reshape(B, -1), _N)
    indices = topk_idx // _NUM_CLASSES
    classes = topk_idx % _NUM_CLASSES
    box_topk = jnp.take_along_axis(box_all, indices[:, :, None], axis=1)
    anc_sel = anchor_boxes[indices]
    scores = jax.nn.sigmoid(topv)

    pad = _NP - _N
    rel = jnp.pad(jnp.transpose(box_topk, (0, 2, 1)), ((0, 0), (0, 0), (0, pad)))
    anc = jnp.pad(jnp.transpose(anc_sel, (0, 2, 1)), ((0, 0), (0, 0), (0, pad)))
    clsf = jnp.pad(classes.astype(jnp.float32)[:, None, :],
                   ((0, 0), (0, 0), (0, pad)))

    boxes_t, keep = pl.pallas_call(
        _det_kernel,
        grid=(B,),
        in_specs=[
            pl.BlockSpec((1, 4, _NP), lambda b: (b, 0, 0)),
            pl.BlockSpec((1, 4, _NP), lambda b: (b, 0, 0)),
            pl.BlockSpec((1, 1, _NP), lambda b: (b, 0, 0)),
        ],
        out_specs=[
            pl.BlockSpec((1, 4, _NP), lambda b: (b, 0, 0)),
            pl.BlockSpec((1, 1, _NP), lambda b: (b, 0, 0)),
        ],
        out_shape=[
            jax.ShapeDtypeStruct((B, 4, _NP), jnp.float32),
            jax.ShapeDtypeStruct((B, 1, _NP), jnp.float32),
        ],
    )(rel, anc, clsf)

    boxes = jnp.transpose(boxes_t[:, :, :_N], (0, 2, 1))
    keepb = keep[:, 0, :_N] > 0.0
    sel_pos = jnp.argsort((~keepb).astype(jnp.int32), axis=1)[:, :_NMAX]
    valid = jnp.take_along_axis(keepb, sel_pos, axis=1)
    b = jnp.take_along_axis(boxes, sel_pos[:, :, None], axis=1)
    s = jnp.take_along_axis(scores, sel_pos, axis=1)
    c = jnp.take_along_axis(classes, sel_pos, axis=1).astype(jnp.float32) + 1.0
    det = jnp.concatenate([
        b[:, :, 0:1], b[:, :, 1:2],
        b[:, :, 2:3] - b[:, :, 0:1], b[:, :, 3:4] - b[:, :, 1:2],
        s[:, :, None], c[:, :, None]], axis=2)
    return jnp.where(valid[:, :, None], det, 0.0)
